# baseline (speedup 1.0000x reference)
import functools

import jax
import jax.numpy as jnp
from jax import lax
from jax.experimental import pallas as pl
from jax.experimental.pallas import tpu as pltpu
from jax.experimental.pallas import tpu_sc as plsc

N = 10000
E = 320000
C = 128
K = 27
NB = 2

NC = 2
NS = 16
NW = NC * NS
CH = 128
NCHUNK = 80
NBUF = 2
E_PAD = NW * NCHUNK * CH
STRIPE = 624
TAIL0 = NS * STRIPE
TAIL = N - TAIL0
ACC_ROWS = N + 16

BN_BLK = 400
NBLK = N // BN_BLK
BN_MM = 1000
NBLK_MM = N // BN_MM


def _matmul_body(x_ref, w_ref, o_ref):
    x = x_ref[...]
    for k in range(K):
        o_ref[k] = jnp.dot(x, w_ref[k], preferred_element_type=jnp.float32)


def _taps_matmul(x, w):
    return pl.pallas_call(
        _matmul_body,
        grid=(NBLK_MM,),
        in_specs=[
            pl.BlockSpec((BN_MM, C), lambda i: (i, 0)),
            pl.BlockSpec((K, C, C), lambda i: (0, 0, 0)),
        ],
        out_specs=pl.BlockSpec((K, BN_MM, C), lambda i: (0, i, 0)),
        out_shape=jax.ShapeDtypeStruct((K, N, C), jnp.float32),
    )(x, w)


def _matmul_bn_body(s_ref, stats_ref, g_ref, b_ref, r_ref, w_ref,
                    o_ref, xc_ref):
    inv_n = 1.0 / N
    m = stats_ref[0:1] * inv_n
    v = stats_ref[1:2] * inv_n - m * m
    inv = lax.rsqrt(v + 1e-5)
    xc = jnp.maximum(
        g_ref[...] * (s_ref[...] - m) * inv + b_ref[...] + r_ref[...], 0.0)
    xc_ref[...] = xc
    for k in range(K):
        o_ref[k] = jnp.dot(xc, w_ref[k], preferred_element_type=jnp.float32)


def _taps_matmul_bn(s, stats, gamma, beta, res, w):
    return pl.pallas_call(
        _matmul_bn_body,
        grid=(NBLK_MM,),
        in_specs=[
            pl.BlockSpec((BN_MM, C), lambda i: (i, 0)),
            pl.BlockSpec((2, C), lambda i: (0, 0)),
            pl.BlockSpec((1, C), lambda i: (0, 0)),
            pl.BlockSpec((1, C), lambda i: (0, 0)),
            pl.BlockSpec((BN_MM, C), lambda i: (i, 0)),
            pl.BlockSpec((K, C, C), lambda i: (0, 0, 0)),
        ],
        out_specs=[
            pl.BlockSpec((K, BN_MM, C), lambda i: (0, i, 0)),
            pl.BlockSpec((BN_MM, C), lambda i: (i, 0)),
        ],
        out_shape=[
            jax.ShapeDtypeStruct((K, N, C), jnp.float32),
            jax.ShapeDtypeStruct((N, C), jnp.float32),
        ],
    )(s, stats, gamma, beta, res, w)


def _sc_body(gidx_hbm, dst_hbm, h2_hbm, z_hbm, out_hbm,
             gidx_v, dst_v, b0, b1, acc, g0, g1, s0, s1):
    bufs = (b0, b1)
    gsems = (g0, g1)
    ssems = (s0, s1)
    c = lax.axis_index("c")
    s = lax.axis_index("s")
    wid = c * NS + s
    row0 = pl.multiple_of(s * STRIPE, 8)
    pltpu.sync_copy(z_hbm.at[pl.ds(row0, STRIPE)],
                    acc.at[pl.ds(row0, STRIPE)])

    @pl.when(s == 0)
    def _():
        pltpu.sync_copy(z_hbm.at[pl.ds(TAIL0, TAIL)],
                        acc.at[pl.ds(TAIL0, TAIL)])

    plsc.subcore_barrier()

    HALF = NCHUNK // 2

    def _wait_g(buf, sem):
        pltpu.make_async_copy(h2_hbm.at[pl.ds(0, CH)], buf, sem).wait()

    def _wait_s(buf, sem):
        pltpu.make_async_copy(buf, acc.at[pl.ds(0, CH)], sem).wait()

    for phase in range(2):
        ph0 = phase * HALF
        pltpu.sync_copy(gidx_hbm.at[wid, pl.ds(ph0, HALF)], gidx_v)
        pltpu.sync_copy(dst_hbm.at[wid, pl.ds(ph0, HALF)], dst_v)
        for k in range(NBUF):
            pltpu.async_copy(h2_hbm.at[gidx_v.at[k]], bufs[k], gsems[k])

        def chunk(i, carry):
            base = NBUF * i
            for k in range(NBUF):
                _wait_g(bufs[k], gsems[k])
                pltpu.async_copy(bufs[k], acc.at[dst_v.at[base + k]],
                                 ssems[k], add=True)
            for k in range(NBUF):
                j = base + k
                _wait_s(bufs[k], ssems[k])

                @pl.when(j + NBUF < HALF)
                def _():
                    pltpu.async_copy(h2_hbm.at[gidx_v.at[j + NBUF]],
                                     bufs[k], gsems[k])

            return carry

        lax.fori_loop(0, HALF // NBUF, chunk, 0)
    plsc.subcore_barrier()
    pltpu.sync_copy(acc.at[pl.ds(row0, STRIPE)],
                    out_hbm.at[c, pl.ds(row0, STRIPE)])

    @pl.when(s == 0)
    def _():
        pltpu.sync_copy(acc.at[pl.ds(TAIL0, TAIL)],
                        out_hbm.at[c, pl.ds(TAIL0, TAIL)])


def _sc_gather_scatter(gidx, dstv, h2flat, zeros):
    mesh = plsc.VectorSubcoreMesh(core_axis_name="c", subcore_axis_name="s")
    return pl.kernel(
        _sc_body,
        out_type=jax.ShapeDtypeStruct((NC, N, C), jnp.float32),
        mesh=mesh,
        scratch_types=[
            pltpu.VMEM((NCHUNK // 2, CH), jnp.int32),
            pltpu.VMEM((NCHUNK // 2, CH), jnp.int32),
        ] + [pltpu.VMEM((CH, C), jnp.float32)] * NBUF + [
            pltpu.VMEM_SHARED((ACC_ROWS, C), jnp.float32),
        ] + [pltpu.SemaphoreType.DMA] * (2 * NBUF),
    )(gidx, dstv, h2flat, zeros)


def _combine_body(p_ref, hc_ref, b_ref, s_ref, stats_ref, acc_ref):
    i = pl.program_id(0)

    @pl.when(i == 0)
    def _():
        acc_ref[...] = jnp.zeros_like(acc_ref)

    s = p_ref[0] + p_ref[1] + hc_ref[0] + b_ref[...]
    s_ref[...] = s
    acc_ref[0:1] += jnp.sum(s, axis=0, keepdims=True)
    acc_ref[1:2] += jnp.sum(s * s, axis=0, keepdims=True)

    @pl.when(i == NBLK - 1)
    def _():
        stats_ref[...] = acc_ref[...]


def _combine(partials, h2, bias):
    return pl.pallas_call(
        _combine_body,
        grid=(NBLK,),
        in_specs=[
            pl.BlockSpec((NC, BN_BLK, C), lambda i: (0, i, 0)),
            pl.BlockSpec((1, BN_BLK, C), lambda i: (K - 1, i, 0)),
            pl.BlockSpec((1, C), lambda i: (0, 0)),
        ],
        out_specs=[
            pl.BlockSpec((BN_BLK, C), lambda i: (i, 0)),
            pl.BlockSpec((2, C), lambda i: (0, 0)),
        ],
        out_shape=[
            jax.ShapeDtypeStruct((N, C), jnp.float32),
            jax.ShapeDtypeStruct((2, C), jnp.float32),
        ],
        scratch_shapes=[pltpu.VMEM((2, C), jnp.float32)],
    )(partials, h2, bias)


def _norm_body(s_ref, stats_ref, g_ref, b_ref, r_ref, o_ref):
    inv_n = 1.0 / N
    m = stats_ref[0:1] * inv_n
    v = stats_ref[1:2] * inv_n - m * m
    inv = lax.rsqrt(v + 1e-5)
    o_ref[...] = jnp.maximum(
        g_ref[...] * (s_ref[...] - m) * inv + b_ref[...] + r_ref[...], 0.0)


def _norm_relu(s, stats, gamma, beta, res):
    return pl.pallas_call(
        _norm_body,
        grid=(NBLK,),
        in_specs=[
            pl.BlockSpec((BN_BLK, C), lambda i: (i, 0)),
            pl.BlockSpec((2, C), lambda i: (0, 0)),
            pl.BlockSpec((1, C), lambda i: (0, 0)),
            pl.BlockSpec((1, C), lambda i: (0, 0)),
            pl.BlockSpec((BN_BLK, C), lambda i: (i, 0)),
        ],
        out_specs=pl.BlockSpec((BN_BLK, C), lambda i: (i, 0)),
        out_shape=jax.ShapeDtypeStruct((N, C), jnp.float32),
    )(s, stats, gamma, beta, res)


def kernel(x, edge_index, offsets, W, bias, gamma, beta):
    src = edge_index[0].astype(jnp.int32)
    dst = edge_index[1].astype(jnp.int32)
    off = offsets.astype(jnp.int32)

    n_pad = E_PAD - E
    pad_ids = jnp.arange(n_pad, dtype=jnp.int32)
    gidx = jnp.concatenate([off * N + src, (pad_ids * 37) % (N * K)])
    dstv = jnp.concatenate([dst, N + (pad_ids % 16)])
    gidx = gidx.reshape(NW, NCHUNK, CH)
    dstv = dstv.reshape(NW, NCHUNK, CH)

    zeros = jnp.zeros((N, C), jnp.float32)

    pending = None
    xc = x
    block_in = x
    for i in range(NB):
        for j in range(2):
            if pending is None:
                h2 = _taps_matmul(xc, W[i, j])
            else:
                h2, xc = _taps_matmul_bn(*pending, W[i, j])
                if j == 0:
                    block_in = xc
            partials = _sc_gather_scatter(gidx, dstv, h2.reshape(K * N, C),
                                          zeros)
            s, stats = _combine(partials, h2, bias[i, j].reshape(1, C))
            res = block_in if j == 1 else zeros
            pending = (s, stats, gamma[i, j].reshape(1, C),
                       beta[i, j].reshape(1, C), res)
    return _norm_relu(*pending)


# speedup vs baseline: 1.1961x; 1.1961x over previous
import functools

import jax
import jax.numpy as jnp
from jax import lax
from jax.experimental import pallas as pl
from jax.experimental.pallas import tpu as pltpu
from jax.experimental.pallas import tpu_sc as plsc

N = 10000
E = 320000
C = 128
K = 27
NB = 2

NC = 2
NS = 16
NW = NC * NS
CH = 128
NCHUNK = 80
NBUF = 2
E_PAD = NW * NCHUNK * CH
STRIPE = 624
TAIL0 = NS * STRIPE
TAIL = N - TAIL0
ACC_ROWS = N + 16

BN_BLK = 400
NBLK = N // BN_BLK
BN_MM = 1000
NBLK_MM = N // BN_MM


def _matmul_body(x_ref, w_ref, o_ref):
    x = x_ref[...]
    for k in range(K):
        o_ref[k] = jnp.dot(x, w_ref[k], preferred_element_type=jnp.float32)


def _taps_matmul(x, w):
    return pl.pallas_call(
        _matmul_body,
        grid=(NBLK_MM,),
        in_specs=[
            pl.BlockSpec((BN_MM, C), lambda i: (i, 0)),
            pl.BlockSpec((K, C, C), lambda i: (0, 0, 0)),
        ],
        out_specs=pl.BlockSpec((K, BN_MM, C), lambda i: (0, i, 0)),
        out_shape=jax.ShapeDtypeStruct((K, N, C), jnp.float32),
    )(x, w)


def _matmul_bn_body(s_ref, stats_ref, g_ref, b_ref, r_ref, w_ref,
                    o_ref, xc_ref):
    inv_n = 1.0 / N
    m = stats_ref[0:1] * inv_n
    v = stats_ref[1:2] * inv_n - m * m
    inv = lax.rsqrt(v + 1e-5)
    xc = jnp.maximum(
        g_ref[...] * (s_ref[...] - m) * inv + b_ref[...] + r_ref[...], 0.0)
    xc_ref[...] = xc
    for k in range(K):
        o_ref[k] = jnp.dot(xc, w_ref[k], preferred_element_type=jnp.float32)


def _taps_matmul_bn(s, stats, gamma, beta, res, w):
    return pl.pallas_call(
        _matmul_bn_body,
        grid=(NBLK_MM,),
        in_specs=[
            pl.BlockSpec((BN_MM, C), lambda i: (i, 0)),
            pl.BlockSpec((2, C), lambda i: (0, 0)),
            pl.BlockSpec((1, C), lambda i: (0, 0)),
            pl.BlockSpec((1, C), lambda i: (0, 0)),
            pl.BlockSpec((BN_MM, C), lambda i: (i, 0)),
            pl.BlockSpec((K, C, C), lambda i: (0, 0, 0)),
        ],
        out_specs=[
            pl.BlockSpec((K, BN_MM, C), lambda i: (0, i, 0)),
            pl.BlockSpec((BN_MM, C), lambda i: (i, 0)),
        ],
        out_shape=[
            jax.ShapeDtypeStruct((K, N, C), jnp.float32),
            jax.ShapeDtypeStruct((N, C), jnp.float32),
        ],
    )(s, stats, gamma, beta, res, w)


def _sc_body(gidx_hbm, dst_hbm, h2_hbm, z_hbm, out_hbm,
             gidx_v, dst_v, b0, b1, acc, g0, g1, s0, s1):
    bufs = (b0, b1)
    gsems = (g0, g1)
    ssems = (s0, s1)
    c = lax.axis_index("c")
    s = lax.axis_index("s")
    wid = c * NS + s
    row0 = pl.multiple_of(s * STRIPE, 8)
    pltpu.sync_copy(z_hbm.at[pl.ds(row0, STRIPE)],
                    acc.at[pl.ds(row0, STRIPE)])

    @pl.when(s == 0)
    def _():
        pltpu.sync_copy(z_hbm.at[pl.ds(TAIL0, TAIL)],
                        acc.at[pl.ds(TAIL0, TAIL)])

    plsc.subcore_barrier()

    HALF = NCHUNK // 2

    def _wait_g(buf, sem):
        pltpu.make_async_copy(h2_hbm.at[pl.ds(0, CH)], buf, sem).wait()

    def _wait_s(buf, sem):
        pltpu.make_async_copy(buf, acc.at[pl.ds(0, CH)], sem).wait()

    for phase in range(2):
        ph0 = phase * HALF
        pltpu.sync_copy(gidx_hbm.at[wid, pl.ds(ph0, HALF)], gidx_v)
        pltpu.sync_copy(dst_hbm.at[wid, pl.ds(ph0, HALF)], dst_v)
        for k in range(NBUF):
            pltpu.async_copy(h2_hbm.at[gidx_v.at[k]], bufs[k], gsems[k])

        def chunk(i, carry):
            base = NBUF * i
            for k in range(NBUF):
                j = base + k
                _wait_g(bufs[k], gsems[k])
                pltpu.sync_copy(bufs[k], acc.at[dst_v.at[j]], add=True)

                @pl.when(j + NBUF < HALF)
                def _():
                    pltpu.async_copy(h2_hbm.at[gidx_v.at[j + NBUF]],
                                     bufs[k], gsems[k])

            return carry

        lax.fori_loop(0, HALF // NBUF, chunk, 0)
    plsc.subcore_barrier()
    pltpu.sync_copy(acc.at[pl.ds(row0, STRIPE)],
                    out_hbm.at[c, pl.ds(row0, STRIPE)])

    @pl.when(s == 0)
    def _():
        pltpu.sync_copy(acc.at[pl.ds(TAIL0, TAIL)],
                        out_hbm.at[c, pl.ds(TAIL0, TAIL)])


def _sc_gather_scatter(gidx, dstv, h2flat, zeros):
    mesh = plsc.VectorSubcoreMesh(core_axis_name="c", subcore_axis_name="s")
    return pl.kernel(
        _sc_body,
        out_type=jax.ShapeDtypeStruct((NC, N, C), jnp.float32),
        mesh=mesh,
        scratch_types=[
            pltpu.VMEM((NCHUNK // 2, CH), jnp.int32),
            pltpu.VMEM((NCHUNK // 2, CH), jnp.int32),
        ] + [pltpu.VMEM((CH, C), jnp.float32)] * NBUF + [
            pltpu.VMEM_SHARED((ACC_ROWS, C), jnp.float32),
        ] + [pltpu.SemaphoreType.DMA] * (2 * NBUF),
    )(gidx, dstv, h2flat, zeros)


def _combine_body(p_ref, hc_ref, b_ref, s_ref, stats_ref, acc_ref):
    i = pl.program_id(0)

    @pl.when(i == 0)
    def _():
        acc_ref[...] = jnp.zeros_like(acc_ref)

    s = p_ref[0] + p_ref[1] + hc_ref[0] + b_ref[...]
    s_ref[...] = s
    acc_ref[0:1] += jnp.sum(s, axis=0, keepdims=True)
    acc_ref[1:2] += jnp.sum(s * s, axis=0, keepdims=True)

    @pl.when(i == NBLK - 1)
    def _():
        stats_ref[...] = acc_ref[...]


def _combine(partials, h2, bias):
    return pl.pallas_call(
        _combine_body,
        grid=(NBLK,),
        in_specs=[
            pl.BlockSpec((NC, BN_BLK, C), lambda i: (0, i, 0)),
            pl.BlockSpec((1, BN_BLK, C), lambda i: (K - 1, i, 0)),
            pl.BlockSpec((1, C), lambda i: (0, 0)),
        ],
        out_specs=[
            pl.BlockSpec((BN_BLK, C), lambda i: (i, 0)),
            pl.BlockSpec((2, C), lambda i: (0, 0)),
        ],
        out_shape=[
            jax.ShapeDtypeStruct((N, C), jnp.float32),
            jax.ShapeDtypeStruct((2, C), jnp.float32),
        ],
        scratch_shapes=[pltpu.VMEM((2, C), jnp.float32)],
    )(partials, h2, bias)


def _norm_body(s_ref, stats_ref, g_ref, b_ref, r_ref, o_ref):
    inv_n = 1.0 / N
    m = stats_ref[0:1] * inv_n
    v = stats_ref[1:2] * inv_n - m * m
    inv = lax.rsqrt(v + 1e-5)
    o_ref[...] = jnp.maximum(
        g_ref[...] * (s_ref[...] - m) * inv + b_ref[...] + r_ref[...], 0.0)


def _norm_relu(s, stats, gamma, beta, res):
    return pl.pallas_call(
        _norm_body,
        grid=(NBLK,),
        in_specs=[
            pl.BlockSpec((BN_BLK, C), lambda i: (i, 0)),
            pl.BlockSpec((2, C), lambda i: (0, 0)),
            pl.BlockSpec((1, C), lambda i: (0, 0)),
            pl.BlockSpec((1, C), lambda i: (0, 0)),
            pl.BlockSpec((BN_BLK, C), lambda i: (i, 0)),
        ],
        out_specs=pl.BlockSpec((BN_BLK, C), lambda i: (i, 0)),
        out_shape=jax.ShapeDtypeStruct((N, C), jnp.float32),
    )(s, stats, gamma, beta, res)


def kernel(x, edge_index, offsets, W, bias, gamma, beta):
    src = edge_index[0].astype(jnp.int32)
    dst = edge_index[1].astype(jnp.int32)
    off = offsets.astype(jnp.int32)

    n_pad = E_PAD - E
    pad_ids = jnp.arange(n_pad, dtype=jnp.int32)
    gidx = jnp.concatenate([off * N + src, (pad_ids * 37) % (N * K)])
    dstv = jnp.concatenate([dst, N + (pad_ids % 16)])
    gidx = gidx.reshape(NW, NCHUNK, CH)
    dstv = dstv.reshape(NW, NCHUNK, CH)

    zeros = jnp.zeros((N, C), jnp.float32)

    pending = None
    xc = x
    block_in = x
    for i in range(NB):
        for j in range(2):
            if pending is None:
                h2 = _taps_matmul(xc, W[i, j])
            else:
                h2, xc = _taps_matmul_bn(*pending, W[i, j])
                if j == 0:
                    block_in = xc
            partials = _sc_gather_scatter(gidx, dstv, h2.reshape(K * N, C),
                                          zeros)
            s, stats = _combine(partials, h2, bias[i, j].reshape(1, C))
            res = block_in if j == 1 else zeros
            pending = (s, stats, gamma[i, j].reshape(1, C),
                       beta[i, j].reshape(1, C), res)
    return _norm_relu(*pending)


# speedup vs baseline: 1.2752x; 1.0662x over previous
import functools

import jax
import jax.numpy as jnp
from jax import lax
from jax.experimental import pallas as pl
from jax.experimental.pallas import tpu as pltpu
from jax.experimental.pallas import tpu_sc as plsc

N = 10000
E = 320000
C = 128
K = 27
NB = 2

NC = 2
NS = 16
NW = NC * NS
CH = 128
NCHUNK = 80
NBUF = 2
E_PAD = NW * NCHUNK * CH
STRIPE = 624
TAIL0 = NS * STRIPE
TAIL = N - TAIL0
ACC_ROWS = N + 16

BN_BLK = 400
NBLK = N // BN_BLK
BN_MM = 1000
NBLK_MM = N // BN_MM


def _matmul_body(x_ref, w_ref, o_ref):
    x = x_ref[...]
    for k in range(K):
        o_ref[k] = jnp.dot(x, w_ref[k], preferred_element_type=jnp.float32)


def _taps_matmul(x, w):
    return pl.pallas_call(
        _matmul_body,
        grid=(NBLK_MM,),
        in_specs=[
            pl.BlockSpec((BN_MM, C), lambda i: (i, 0)),
            pl.BlockSpec((K, C, C), lambda i: (0, 0, 0)),
        ],
        out_specs=pl.BlockSpec((K, BN_MM, C), lambda i: (0, i, 0)),
        out_shape=jax.ShapeDtypeStruct((K, N, C), jnp.float32),
    )(x, w)


def _phase_maps(row_block):
    prev = lambda i: jnp.minimum(i, NBLK_MM - 1)
    cur = lambda i: jnp.maximum(i - NBLK_MM, 0)
    return prev, cur


def _combine_mm_body(p_ref, hc_ref, b_ref, g_ref, bt_ref, r_ref, w_ref,
                     o_ref, xc_ref, s_scr, acc_ref):
    i = pl.program_id(0)

    @pl.when(i == 0)
    def _():
        acc_ref[...] = jnp.zeros_like(acc_ref)

    @pl.when(i < NBLK_MM)
    def _():
        s = p_ref[0] + p_ref[1] + hc_ref[0] + b_ref[...]
        s_scr[pl.ds(i * BN_MM, BN_MM)] = s
        acc_ref[0:1] += jnp.sum(s, axis=0, keepdims=True)
        acc_ref[1:2] += jnp.sum(s * s, axis=0, keepdims=True)

    @pl.when(i >= NBLK_MM)
    def _():
        inv_n = 1.0 / N
        m = acc_ref[0:1] * inv_n
        v = acc_ref[1:2] * inv_n - m * m
        inv = lax.rsqrt(v + 1e-5)
        s = s_scr[pl.ds((i - NBLK_MM) * BN_MM, BN_MM)]
        xc = jnp.maximum(
            g_ref[...] * (s - m) * inv + bt_ref[...] + r_ref[...], 0.0)
        xc_ref[...] = xc
        for k in range(K):
            o_ref[k] = jnp.dot(xc, w_ref[k],
                               preferred_element_type=jnp.float32)


def _combine_mm(partials, h2, bias, gamma, beta, res, w):
    prev, cur = _phase_maps(BN_MM)
    return pl.pallas_call(
        _combine_mm_body,
        grid=(2 * NBLK_MM,),
        in_specs=[
            pl.BlockSpec((NC, BN_MM, C), lambda i: (0, prev(i), 0)),
            pl.BlockSpec((1, BN_MM, C), lambda i: (K - 1, prev(i), 0)),
            pl.BlockSpec((1, C), lambda i: (0, 0)),
            pl.BlockSpec((1, C), lambda i: (0, 0)),
            pl.BlockSpec((1, C), lambda i: (0, 0)),
            pl.BlockSpec((BN_MM, C), lambda i: (cur(i), 0)),
            pl.BlockSpec((K, C, C), lambda i: (0, 0, 0)),
        ],
        out_specs=[
            pl.BlockSpec((K, BN_MM, C), lambda i: (0, cur(i), 0)),
            pl.BlockSpec((BN_MM, C), lambda i: (cur(i), 0)),
        ],
        out_shape=[
            jax.ShapeDtypeStruct((K, N, C), jnp.float32),
            jax.ShapeDtypeStruct((N, C), jnp.float32),
        ],
        scratch_shapes=[
            pltpu.VMEM((N, C), jnp.float32),
            pltpu.VMEM((2, C), jnp.float32),
        ],
    )(partials, h2, bias, gamma, beta, res, w)


def _combine_norm_body(p_ref, hc_ref, b_ref, g_ref, bt_ref, r_ref,
                       xc_ref, s_scr, acc_ref):
    i = pl.program_id(0)

    @pl.when(i == 0)
    def _():
        acc_ref[...] = jnp.zeros_like(acc_ref)

    @pl.when(i < NBLK_MM)
    def _():
        s = p_ref[0] + p_ref[1] + hc_ref[0] + b_ref[...]
        s_scr[pl.ds(i * BN_MM, BN_MM)] = s
        acc_ref[0:1] += jnp.sum(s, axis=0, keepdims=True)
        acc_ref[1:2] += jnp.sum(s * s, axis=0, keepdims=True)

    @pl.when(i >= NBLK_MM)
    def _():
        inv_n = 1.0 / N
        m = acc_ref[0:1] * inv_n
        v = acc_ref[1:2] * inv_n - m * m
        inv = lax.rsqrt(v + 1e-5)
        s = s_scr[pl.ds((i - NBLK_MM) * BN_MM, BN_MM)]
        xc_ref[...] = jnp.maximum(
            g_ref[...] * (s - m) * inv + bt_ref[...] + r_ref[...], 0.0)


def _combine_norm(partials, h2, bias, gamma, beta, res):
    prev, cur = _phase_maps(BN_MM)
    return pl.pallas_call(
        _combine_norm_body,
        grid=(2 * NBLK_MM,),
        in_specs=[
            pl.BlockSpec((NC, BN_MM, C), lambda i: (0, prev(i), 0)),
            pl.BlockSpec((1, BN_MM, C), lambda i: (K - 1, prev(i), 0)),
            pl.BlockSpec((1, C), lambda i: (0, 0)),
            pl.BlockSpec((1, C), lambda i: (0, 0)),
            pl.BlockSpec((1, C), lambda i: (0, 0)),
            pl.BlockSpec((BN_MM, C), lambda i: (cur(i), 0)),
        ],
        out_specs=pl.BlockSpec((BN_MM, C), lambda i: (cur(i), 0)),
        out_shape=jax.ShapeDtypeStruct((N, C), jnp.float32),
        scratch_shapes=[
            pltpu.VMEM((N, C), jnp.float32),
            pltpu.VMEM((2, C), jnp.float32),
        ],
    )(partials, h2, bias, gamma, beta, res)


def _sc_body(gidx_hbm, dst_hbm, h2_hbm, z_hbm, out_hbm,
             gidx_v, dst_v, b0, b1, acc, g0, g1, s0, s1):
    bufs = (b0, b1)
    gsems = (g0, g1)
    ssems = (s0, s1)
    c = lax.axis_index("c")
    s = lax.axis_index("s")
    wid = c * NS + s
    row0 = pl.multiple_of(s * STRIPE, 8)
    pltpu.sync_copy(z_hbm.at[pl.ds(row0, STRIPE)],
                    acc.at[pl.ds(row0, STRIPE)])

    @pl.when(s == 0)
    def _():
        pltpu.sync_copy(z_hbm.at[pl.ds(TAIL0, TAIL)],
                        acc.at[pl.ds(TAIL0, TAIL)])

    plsc.subcore_barrier()

    HALF = NCHUNK // 2

    def _wait_g(buf, sem):
        pltpu.make_async_copy(h2_hbm.at[pl.ds(0, CH)], buf, sem).wait()

    def _wait_s(buf, sem):
        pltpu.make_async_copy(buf, acc.at[pl.ds(0, CH)], sem).wait()

    for phase in range(2):
        ph0 = phase * HALF
        pltpu.sync_copy(gidx_hbm.at[wid, pl.ds(ph0, HALF)], gidx_v)
        pltpu.sync_copy(dst_hbm.at[wid, pl.ds(ph0, HALF)], dst_v)
        for k in range(NBUF):
            pltpu.async_copy(h2_hbm.at[gidx_v.at[k]], bufs[k], gsems[k])

        def chunk(i, carry):
            base = NBUF * i
            for k in range(NBUF):
                j = base + k
                _wait_g(bufs[k], gsems[k])
                pltpu.sync_copy(bufs[k], acc.at[dst_v.at[j]], add=True)

                @pl.when(j + NBUF < HALF)
                def _():
                    pltpu.async_copy(h2_hbm.at[gidx_v.at[j + NBUF]],
                                     bufs[k], gsems[k])

            return carry

        lax.fori_loop(0, HALF // NBUF, chunk, 0)
    plsc.subcore_barrier()
    pltpu.sync_copy(acc.at[pl.ds(row0, STRIPE)],
                    out_hbm.at[c, pl.ds(row0, STRIPE)])

    @pl.when(s == 0)
    def _():
        pltpu.sync_copy(acc.at[pl.ds(TAIL0, TAIL)],
                        out_hbm.at[c, pl.ds(TAIL0, TAIL)])


def _sc_gather_scatter(gidx, dstv, h2flat, zeros):
    mesh = plsc.VectorSubcoreMesh(core_axis_name="c", subcore_axis_name="s")
    return pl.kernel(
        _sc_body,
        out_type=jax.ShapeDtypeStruct((NC, N, C), jnp.float32),
        mesh=mesh,
        scratch_types=[
            pltpu.VMEM((NCHUNK // 2, CH), jnp.int32),
            pltpu.VMEM((NCHUNK // 2, CH), jnp.int32),
        ] + [pltpu.VMEM((CH, C), jnp.float32)] * NBUF + [
            pltpu.VMEM_SHARED((ACC_ROWS, C), jnp.float32),
        ] + [pltpu.SemaphoreType.DMA] * (2 * NBUF),
    )(gidx, dstv, h2flat, zeros)


def kernel(x, edge_index, offsets, W, bias, gamma, beta):
    src = edge_index[0].astype(jnp.int32)
    dst = edge_index[1].astype(jnp.int32)
    off = offsets.astype(jnp.int32)

    n_pad = E_PAD - E
    pad_ids = jnp.arange(n_pad, dtype=jnp.int32)
    gidx = jnp.concatenate([off * N + src, (pad_ids * 37) % (N * K)])
    dstv = jnp.concatenate([dst, N + (pad_ids % 16)])
    gidx = gidx.reshape(NW, NCHUNK, CH)
    dstv = dstv.reshape(NW, NCHUNK, CH)

    zeros = jnp.zeros((N, C), jnp.float32)

    pending = None
    xc = x
    block_in = x
    for i in range(NB):
        for j in range(2):
            if pending is None:
                h2 = _taps_matmul(xc, W[i, j])
            else:
                h2, xc = _combine_mm(*pending, W[i, j])
                if j == 0:
                    block_in = xc
            partials = _sc_gather_scatter(gidx, dstv, h2.reshape(K * N, C),
                                          zeros)
            res = block_in if j == 1 else zeros
            pending = (partials, h2, bias[i, j].reshape(1, C),
                       gamma[i, j].reshape(1, C), beta[i, j].reshape(1, C),
                       res)
    return _combine_norm(*pending)
